# Initial kernel scaffold; baseline (speedup 1.0000x reference)
import jax
import jax.numpy as jnp
from jax.experimental import pallas as pl


def kernel(features_tensor, neighbor_idx, community_idx):
    raise NotImplementedError("write your pallas kernel here")


# speedup vs baseline: 1.5092x; 1.5092x over previous
import functools

import jax
import jax.numpy as jnp
from jax import lax
from jax.experimental import pallas as pl
from jax.experimental.pallas import tpu as pltpu
from jax.experimental.pallas import tpu_sc as plsc

N, D, DEG, C, S = 10000, 128, 32, 4, 8
D2 = 2 * D
NC, NS, L = 2, 16, 16
NW = NC * NS
NPW = 320
NPAD = NW * NPW

CH1 = 4
ITERS1 = NPW // CH1

def _sc_layer1_body(x_hbm, cidx_hbm, h1_hbm, idx_v, rows_v, self_v, h1_v, sem):
    wid = lax.axis_index("s") * NC + lax.axis_index("c")
    wbase = wid * NPW

    def step(g, _):
        base = wbase + g * CH1
        pltpu.sync_copy(cidx_hbm.at[pl.ds(base * DEG, CH1 * DEG)], idx_v)
        cp = pltpu.async_copy(x_hbm.at[idx_v], rows_v, sem)
        pltpu.sync_copy(x_hbm.at[pl.ds(base, CH1)], self_v)
        cp.wait()
        for b in range(CH1):
            for k in range(D // L):
                sl = pl.ds(k * L, L)
                h1_v[b, sl] = jnp.maximum(self_v[b, sl], 0.0)
                m = None
                for c in range(C):
                    r0 = b * DEG + c * S
                    acc = rows_v[r0, sl]
                    for s in range(1, S):
                        acc = acc + rows_v[r0 + s, sl]
                    m = acc if m is None else jnp.maximum(m, acc)
                h1_v[b, pl.ds(D + k * L, L)] = jnp.maximum(m * (1.0 / S), 0.0)
        pltpu.sync_copy(h1_v, h1_hbm.at[pl.ds(base, CH1)])
        return _

    lax.fori_loop(0, ITERS1, step, None)


CH2 = 2
ITERS2 = NPW // CH2


def _sc_layer2_body(h1_hbm, cidx_hbm, nidx_hbm, nf_hbm, cent_hbm, idx_v, rows_v,
                    cent_v, sem):
    wid = lax.axis_index("s") * NC + lax.axis_index("c")
    wbase = wid * NPW

    def step(g, _):
        base = wbase + g * CH2
        pltpu.sync_copy(cidx_hbm.at[pl.ds(base * DEG, CH2 * DEG)],
                        idx_v.at[pl.ds(0, CH2 * DEG)])
        pltpu.sync_copy(nidx_hbm.at[pl.ds(base * DEG, CH2 * DEG)],
                        idx_v.at[pl.ds(CH2 * DEG, CH2 * DEG)])
        cp = pltpu.async_copy(h1_hbm.at[idx_v], rows_v, sem)
        cp.wait()
        pltpu.sync_copy(rows_v.at[pl.ds(CH2 * DEG, CH2 * DEG)],
                        nf_hbm.at[pl.ds(base * DEG, CH2 * DEG)])
        for b in range(CH2):
            for c in range(C):
                r0 = b * DEG + c * S
                for k in range(D2 // L):
                    sl = pl.ds(k * L, L)
                    acc = rows_v[r0, sl]
                    for s in range(1, S):
                        acc = acc + rows_v[r0 + s, sl]
                    cent_v[b * C + c, sl] = acc * (1.0 / S)
        pltpu.sync_copy(cent_v, cent_hbm.at[pl.ds(base * C, CH2 * C)])
        return _

    lax.fori_loop(0, ITERS2, step, None)


@functools.lru_cache(maxsize=None)
def _build_sc_kernels():
    mesh = plsc.VectorSubcoreMesh(core_axis_name="c", subcore_axis_name="s")
    k1 = pl.kernel(
        _sc_layer1_body,
        out_type=jax.ShapeDtypeStruct((NPAD, D2), jnp.float32),
        mesh=mesh,
        scratch_types=[
            pltpu.VMEM((CH1 * DEG,), jnp.int32),
            pltpu.VMEM((CH1 * DEG, D), jnp.float32),
            pltpu.VMEM((CH1, D), jnp.float32),
            pltpu.VMEM((CH1, D2), jnp.float32),
            pltpu.SemaphoreType.DMA,
        ],
    )
    k2 = pl.kernel(
        _sc_layer2_body,
        out_type=[
            jax.ShapeDtypeStruct((NPAD * DEG, D2), jnp.float32),
            jax.ShapeDtypeStruct((NPAD * C, D2), jnp.float32),
        ],
        mesh=mesh,
        scratch_types=[
            pltpu.VMEM((CH2 * 2 * DEG,), jnp.int32),
            pltpu.VMEM((CH2 * 2 * DEG, D2), jnp.float32),
            pltpu.VMEM((CH2 * C, D2), jnp.float32),
            pltpu.SemaphoreType.DMA,
        ],
    )
    return k1, k2


BN = 64


def _tc_kmeans_body(nf_ref, cent_ref, out_ref):
    nf = jnp.reshape(nf_ref[...], (BN, DEG, D2))
    cent0 = jnp.reshape(cent_ref[...], (BN, C, D2))

    def assign(cent_list):
        lab = jnp.zeros((BN, DEG), jnp.int32)
        best = None
        for c in range(C):
            diff = nf - cent_list[c][:, None, :]
            dc = jnp.sum(diff * diff, axis=-1)
            if best is None:
                best = dc
            else:
                m = dc < best
                lab = jnp.where(m, c, lab)
                best = jnp.where(m, dc, best)
        return lab

    def cluster_sums(lab):
        sums, cnts = [], []
        for c in range(C):
            mask = (lab == c)
            cnts.append(jnp.sum(jnp.where(mask, 1.0, 0.0), axis=-1))
            sums.append(jnp.sum(jnp.where(mask[:, :, None], nf, 0.0), axis=1))
        return sums, cnts

    cents = [cent0[:, c, :] for c in range(C)]
    lab0 = assign(cents)
    sums0, cnt0 = cluster_sums(lab0)
    new_cents = []
    for c in range(C):
        mean_c = sums0[c] / jnp.maximum(cnt0[c], 1.0)[:, None]
        new_cents.append(jnp.where((cnt0[c] > 0.0)[:, None], mean_c, cents[c]))
    lab1 = assign(new_cents)
    sums1, cnt1 = cluster_sums(lab1)
    t = None
    for c in range(C):
        agg = sums1[c] / jnp.maximum(cnt1[c], 1.0)[:, None]
        t = agg if t is None else jnp.maximum(t, agg)
    out_ref[...] = jnp.maximum(t, 0.0)


def _tc_kmeans(nf, cent):
    return pl.pallas_call(
        _tc_kmeans_body,
        grid=(NPAD // BN,),
        in_specs=[
            pl.BlockSpec((BN * DEG, D2), lambda i: (i, 0)),
            pl.BlockSpec((BN * C, D2), lambda i: (i, 0)),
        ],
        out_specs=pl.BlockSpec((BN, D2), lambda i: (i, 0)),
        out_shape=jax.ShapeDtypeStruct((NPAD, D2), jnp.float32),
    )(nf, cent)


def kernel(features_tensor, neighbor_idx, community_idx):
    x = features_tensor.astype(jnp.float32)
    pad = NPAD - N
    x_p = jnp.pad(x, ((0, pad), (0, 0)))
    fill = (jnp.arange(pad * DEG, dtype=jnp.int32) * 37) % N
    cidx = jnp.concatenate(
        [community_idx.astype(jnp.int32).reshape(N * C * S), fill])
    nidx = jnp.concatenate(
        [neighbor_idx.astype(jnp.int32).reshape(N * DEG), fill])

    sc1, sc2 = _build_sc_kernels()
    h1 = sc1(x_p, cidx)
    nf, cent = sc2(h1, cidx, nidx)
    t = _tc_kmeans(nf, cent)

    h1c = h1[:N]
    final = jnp.concatenate([h1c, h1c, t[:N]], axis=-1)
    return (final, final)
